# baseline (speedup 1.0000x reference)
import functools

import jax
import jax.numpy as jnp
from jax import lax
from jax.experimental import pallas as pl
from jax.experimental.pallas import tpu as pltpu
from jax.experimental.pallas import tpu_sc as plsc

D = 128
CH = 200
NBUF = 2
SUP = NBUF * CH
SCH = 128


@functools.lru_cache(maxsize=None)
def _make_gather(n_rows: int, n_tab: int):
    info = plsc.get_sparse_core_info()
    nc, ns = info.num_cores, info.num_subcores
    nw = nc * ns
    assert n_rows % (nw * SUP) == 0
    per_w = n_rows // nw
    n_super = per_w // SUP
    assert n_tab % (ns * SCH) == 0
    tab_per_s = n_tab // ns

    mesh = plsc.VectorSubcoreMesh(core_axis_name="c", subcore_axis_name="s")

    @functools.partial(
        pl.kernel,
        out_type=jax.ShapeDtypeStruct((n_rows, D), jnp.float32),
        mesh=mesh,
        scratch_types=[
            pltpu.VMEM((2 * SUP,), jnp.int32),
            pltpu.VMEM((NBUF, CH, D), jnp.float32),
            pltpu.VMEM_SHARED((n_tab, D), jnp.float32),
            pltpu.SemaphoreType.DMA((NBUF,)),
            pltpu.SemaphoreType.DMA((NBUF,)),
        ],
    )
    def k(tab_hbm, idx_hbm, out_hbm, idx_v, rows, stab, gsem, ssem):
        wid = lax.axis_index("s") * nc + lax.axis_index("c")
        sid = lax.axis_index("s")
        base = wid * per_w

        for j in range(tab_per_s // SCH):
            off = sid * tab_per_s + j * SCH
            pltpu.sync_copy(tab_hbm.at[pl.ds(off, SCH)], rows.at[0, pl.ds(0, SCH)])
            pltpu.sync_copy(rows.at[0, pl.ds(0, SCH)], stab.at[pl.ds(off, SCH)])
        pltpu.sync_copy(idx_hbm.at[pl.ds(base, SUP)], idx_v.at[pl.ds(0, SUP)])
        plsc.subcore_barrier()

        for b in range(NBUF):
            pltpu.async_copy(
                stab.at[idx_v.at[pl.ds(b * CH, CH)]], rows.at[b], gsem.at[b]
            )

        def sup(s, carry):
            nxt = ((s + 1) % 2) * SUP
            pltpu.sync_copy(
                idx_hbm.at[pl.ds(base + (s + 1) * SUP, SUP)],
                idx_v.at[pl.ds(nxt, SUP)],
            )
            for b in range(NBUF):
                pltpu.make_async_copy(
                    stab.at[pl.ds(0, CH)], rows.at[b], gsem.at[b]
                ).wait()
                pltpu.async_copy(
                    rows.at[b],
                    out_hbm.at[pl.ds(base + s * SUP + b * CH, CH)],
                    ssem.at[b],
                )
            for b in range(NBUF):
                pltpu.make_async_copy(
                    rows.at[b], out_hbm.at[pl.ds(0, CH)], ssem.at[b]
                ).wait()
                pltpu.async_copy(
                    stab.at[idx_v.at[pl.ds(nxt + b * CH, CH)]],
                    rows.at[b],
                    gsem.at[b],
                )
            return carry

        lax.fori_loop(0, n_super - 1, sup, 0)

        last = base + (n_super - 1) * SUP
        for b in range(NBUF):
            pltpu.make_async_copy(
                stab.at[pl.ds(0, CH)], rows.at[b], gsem.at[b]
            ).wait()
            pltpu.async_copy(
                rows.at[b], out_hbm.at[pl.ds(last + b * CH, CH)], ssem.at[b]
            )
        for b in range(NBUF):
            pltpu.make_async_copy(
                rows.at[b], out_hbm.at[pl.ds(0, CH)], ssem.at[b]
            ).wait()

    return k


def kernel(indices, pe):
    b, kk = indices.shape
    table = pe[0]
    idx = indices.reshape(-1).astype(jnp.int32)
    out = _make_gather(b * kk, table.shape[0])(table, idx)
    return out.reshape(b, kk, D)
